# Initial kernel scaffold; baseline (speedup 1.0000x reference)
import jax
import jax.numpy as jnp
from jax.experimental import pallas as pl


def kernel(pos, batch, W, b):
    raise NotImplementedError("write your pallas kernel here")


# speedup vs baseline: 2.6155x; 2.6155x over previous
import functools

import jax
import jax.numpy as jnp
from jax import lax
from jax.experimental import pallas as pl
from jax.experimental.pallas import tpu as pltpu
from jax.experimental.pallas import tpu_sc as plsc

NSEG = 4096
NCORE = 2
NSUB = 16
NW = NCORE * NSUB
LANES = 16
WIN = 4000


def _iota():
    return lax.iota(jnp.int32, LANES)


def _phase1_body(n_pts, pos_hbm, batch_hbm, part_hbm, pos_win, batch_win,
                 acc, red, shared):
    cid = lax.axis_index("c")
    sid = lax.axis_index("s")
    wid = sid * NCORE + cid
    pts_per = n_pts // NW
    nwin = pts_per // WIN

    iota = _iota()
    zero16 = jnp.zeros((LANES,), jnp.float32)

    def zero_body(i, _):
        acc[pl.ds(i * LANES, LANES)] = zero16
        return 0
    lax.fori_loop(0, NSEG, zero_body, 0)

    def win_body(w, _):
        base = wid * pts_per + w * WIN
        pltpu.sync_copy(pos_hbm.at[pl.ds(3 * base, 3 * WIN)],
                        pos_win.at[pl.ds(0, 3 * WIN)])
        pltpu.sync_copy(batch_hbm.at[pl.ds(base, WIN)], batch_win)

        def grp_body(i, _):
            l_iota = _iota()
            l_q4 = lax.shift_right_logical(l_iota, 2)
            l_r4 = lax.bitwise_and(l_iota, 3)
            l_mask3 = l_r4 == 3
            l_v3 = 3 * l_q4 + l_r4
            for u in range(4):
                g = i * 4 + u
                ids4 = plsc.load_gather(batch_win, [l_q4 + 4 * g])
                addr = ids4 * LANES + l_iota
                vals = plsc.load_gather(pos_win, [l_v3 + 12 * g])
                vals = jnp.where(l_mask3, 1.0, vals)
                plsc.addupdate_scatter(acc, [addr], vals)
            return 0
        lax.fori_loop(0, WIN // LANES, grp_body, 0)
        return 0
    lax.fori_loop(0, nwin, win_body, 0)

    ix8 = iota ^ 8
    ix4 = iota ^ 4
    ix12 = iota ^ 12
    mlow = iota < 4

    def red_body(s, _):
        b = s * LANES
        v = acc[pl.ds(b, LANES)]
        g8 = plsc.load_gather(acc, [b + ix8])
        g4 = plsc.load_gather(acc, [b + ix4])
        g12 = plsc.load_gather(acc, [b + ix12])
        v4 = (v + g8) + (g4 + g12)
        plsc.store_scatter(red, [4 * s + iota], v4, mask=mlow)
        return 0
    lax.fori_loop(0, NSEG, red_body, 0)

    pltpu.sync_copy(red.at[pl.ds(0, 4 * NSEG)], shared.at[sid])
    plsc.subcore_barrier()

    seg = 4 * NSEG // NSUB
    for r in range(NSUB):
        pltpu.sync_copy(shared.at[r, pl.ds(sid * seg, seg)],
                        acc.at[pl.ds(r * seg, seg)])

    def sum_body(j, _):
        t = zero16
        for r in range(NSUB):
            t = t + acc[pl.ds(r * seg + j * LANES, LANES)]
        red[pl.ds(j * LANES, LANES)] = t
        return 0
    lax.fori_loop(0, seg // LANES, sum_body, 0)

    pltpu.sync_copy(red.at[pl.ds(0, seg)],
                    part_hbm.at[pl.ds(cid * 4 * NSEG + sid * seg, seg)])


def _phase2_body(n_pts, pos_hbm, batch_hbm, part_hbm, par_hbm, out_hbm,
                 pos_win, batch_win, out_win, buf_a, buf_b, hbuf, pbuf):
    cid = lax.axis_index("c")
    sid = lax.axis_index("s")
    wid = sid * NCORE + cid
    pts_per = n_pts // NW
    nwin = pts_per // WIN

    iota = _iota()
    r4 = lax.bitwise_and(iota, 3)
    mask3 = r4 == 3
    or3 = iota | 3
    ix1 = iota ^ 1
    ix2 = iota ^ 2
    ix3 = iota ^ 3

    pltpu.sync_copy(part_hbm.at[pl.ds(0, 4 * NSEG)], buf_a)
    pltpu.sync_copy(part_hbm.at[pl.ds(4 * NSEG, 4 * NSEG)], buf_b)
    pltpu.sync_copy(par_hbm, pbuf)
    av = pbuf[pl.ds(0, LANES)]
    c0v = pbuf[pl.ds(LANES, LANES)]

    def h_body(i, _):
        b = i * LANES
        v = buf_a[pl.ds(b, LANES)] + buf_b[pl.ds(b, LANES)]
        buf_a[pl.ds(b, LANES)] = v
        cnt = plsc.load_gather(buf_a, [b + or3])
        c = v / jnp.maximum(cnt, 1.0)
        csq = jnp.where(mask3, 0.0, c * c)
        buf_b[pl.ds(b, LANES)] = csq
        s2 = ((csq + plsc.load_gather(buf_b, [b + ix1]))
              + (plsc.load_gather(buf_b, [b + ix2])
                 + plsc.load_gather(buf_b, [b + ix3])))
        h = jnp.where(mask3, av * s2 + c0v, (-2.0 * av) * c)
        hbuf[pl.ds(b, LANES)] = h
        return 0
    lax.fori_loop(0, 4 * NSEG // LANES, h_body, 0)

    def win_body(w, _):
        base = wid * pts_per + w * WIN
        pltpu.sync_copy(pos_hbm.at[pl.ds(3 * base, 3 * WIN)],
                        pos_win.at[pl.ds(0, 3 * WIN)])
        pltpu.sync_copy(batch_hbm.at[pl.ds(base, WIN)], batch_win)

        def pt_body(i, _):
            l_iota3 = 3 * _iota()
            off = i * LANES
            idv = batch_win[pl.ds(off, LANES)]
            pidx = 48 * i + l_iota3
            xs = plsc.load_gather(pos_win, [pidx])
            ys = plsc.load_gather(pos_win, [pidx + 1])
            zs = plsc.load_gather(pos_win, [pidx + 2])
            hidx = idv * 4
            hx = plsc.load_gather(hbuf, [hidx])
            hy = plsc.load_gather(hbuf, [hidx + 1])
            hz = plsc.load_gather(hbuf, [hidx + 2])
            hw = plsc.load_gather(hbuf, [hidx + 3])
            s2p = xs * xs + ys * ys + zs * zs
            dot = xs * hx + ys * hy + zs * hz
            logit = av * s2p + dot + hw
            sig = 1.0 / (1.0 + jnp.exp(-logit))
            out_win[pl.ds(off, LANES)] = sig + 1e-4
            return 0
        lax.fori_loop(0, WIN // LANES, pt_body, 0)

        pltpu.sync_copy(out_win, out_hbm.at[pl.ds(base, WIN)])
        return 0
    lax.fori_loop(0, nwin, win_body, 0)


def kernel(pos, batch, W, b):
    n = pos.shape[0]
    pos_flat = pos.reshape(-1)
    w00 = jnp.full((LANES,), W[0, 0], jnp.float32)
    c0 = jnp.full((LANES,), W[0, 1] + b[0], jnp.float32)
    params = jnp.concatenate([w00, c0])

    mesh = plsc.VectorSubcoreMesh(core_axis_name="c", subcore_axis_name="s")

    cparams = pltpu.CompilerParams(needs_layout_passes=False)

    k1 = functools.partial(
        pl.kernel,
        out_type=jax.ShapeDtypeStruct((NCORE * 4 * NSEG,), jnp.float32),
        mesh=mesh,
        compiler_params=cparams,
        scratch_types=[
            pltpu.VMEM((3 * WIN + 8,), jnp.float32),
            pltpu.VMEM((WIN,), jnp.int32),
            pltpu.VMEM((NSEG * LANES,), jnp.float32),
            pltpu.VMEM((4 * NSEG + 16,), jnp.float32),
            pltpu.VMEM_SHARED((NSUB, 4 * NSEG), jnp.float32),
        ],
    )(functools.partial(_phase1_body, n))
    part = k1(pos_flat, batch)

    k2 = functools.partial(
        pl.kernel,
        out_type=jax.ShapeDtypeStruct((n,), jnp.float32),
        mesh=mesh,
        compiler_params=cparams,
        scratch_types=[
            pltpu.VMEM((3 * WIN + 8,), jnp.float32),
            pltpu.VMEM((WIN,), jnp.int32),
            pltpu.VMEM((WIN,), jnp.float32),
            pltpu.VMEM((4 * NSEG,), jnp.float32),
            pltpu.VMEM((4 * NSEG,), jnp.float32),
            pltpu.VMEM((4 * NSEG + 16,), jnp.float32),
            pltpu.VMEM((2 * LANES,), jnp.float32),
        ],
    )(functools.partial(_phase2_body, n))
    return k2(pos_flat, batch, part, params)
